# baseline (device time: 1439728 ns/iter reference)
import jax
import jax.numpy as jnp
from jax import lax
from jax.experimental import pallas as pl
from jax.experimental.pallas import tpu as pltpu

N_DEV = 16


def kernel(x, w_mat, scale_x, scale_w):
    k_full, k_per = x.shape
    _, n = w_mat.shape
    m_per = k_full // N_DEV

    def body(x_ref, w_ref, sx_ref, sw_ref, out_ref,
             wbf_ref, comm_ref, send_sems, recv_sems, credit_sem):
        my = lax.axis_index("i")
        left = lax.rem(my + N_DEV - 1, N_DEV)
        right = lax.rem(my + 1, N_DEV)

        barrier_sem = pltpu.get_barrier_semaphore()
        for nbr in (left, right):
            pl.semaphore_signal(barrier_sem, inc=1, device_id=(nbr,),
                                device_id_type=pl.DeviceIdType.MESH)
        pl.semaphore_wait(barrier_sem, 2)

        wbf_ref[...] = w_ref[...].astype(jnp.bfloat16)

        def partial(c):
            xa = x_ref[pl.ds(c * m_per, m_per), :].astype(jnp.bfloat16)
            return jnp.dot(xa, wbf_ref[...], preferred_element_type=jnp.float32)

        comm_ref[0, :, :] = partial(left)

        for s in range(N_DEV - 1):
            send_slot = s % 2
            recv_slot = (s + 1) % 2
            if s >= 1:
                pl.semaphore_wait(credit_sem, 1)
            rdma = pltpu.make_async_remote_copy(
                src_ref=comm_ref.at[send_slot],
                dst_ref=comm_ref.at[recv_slot],
                send_sem=send_sems.at[send_slot],
                recv_sem=recv_sems.at[recv_slot],
                device_id=(right,),
                device_id_type=pl.DeviceIdType.MESH,
            )
            rdma.start()
            rdma.wait()
            if s < N_DEV - 2:
                pl.semaphore_signal(credit_sem, inc=1, device_id=(left,),
                                    device_id_type=pl.DeviceIdType.MESH)
                c = lax.rem(my + 2 * N_DEV - 2 - s, N_DEV)
                comm_ref[recv_slot, :, :] = comm_ref[recv_slot, :, :] + partial(c)
            else:
                acc = comm_ref[recv_slot, :, :] + partial(my)
                scale = sx_ref[0] * sw_ref[0]
                out_ref[...] = jnp.maximum(acc * scale, 0.0)

    return pl.pallas_call(
        body,
        out_shape=jax.ShapeDtypeStruct((m_per, n), jnp.float32),
        in_specs=[
            pl.BlockSpec(memory_space=pltpu.VMEM),
            pl.BlockSpec(memory_space=pltpu.VMEM),
            pl.BlockSpec(memory_space=pltpu.SMEM),
            pl.BlockSpec(memory_space=pltpu.SMEM),
        ],
        out_specs=pl.BlockSpec(memory_space=pltpu.VMEM),
        scratch_shapes=[
            pltpu.VMEM((k_per, n), jnp.bfloat16),
            pltpu.VMEM((2, m_per, n), jnp.float32),
            pltpu.SemaphoreType.DMA((2,)),
            pltpu.SemaphoreType.DMA((2,)),
            pltpu.SemaphoreType.REGULAR,
        ],
        compiler_params=pltpu.CompilerParams(collective_id=0),
    )(x, w_mat, scale_x, scale_w)


# device time: 428360 ns/iter; 3.3610x vs baseline; 3.3610x over previous
import jax
import jax.numpy as jnp
from jax import lax
from jax.experimental import pallas as pl
from jax.experimental.pallas import tpu as pltpu

N_DEV = 16


def kernel(x, w_mat, scale_x, scale_w):
    k_full, k_per = x.shape
    _, n = w_mat.shape
    m_per = k_full // N_DEV
    nh = n // 2

    def body(x_ref, w_ref, sx_ref, sw_ref, out_ref,
             wbf_ref, commR, commL, pbR, pbL,
             ssR, rsR, ssL, rsL, credits):
        my = lax.axis_index("i")
        left = lax.rem(my + N_DEV - 1, N_DEV)
        right = lax.rem(my + 1, N_DEV)

        barrier_sem = pltpu.get_barrier_semaphore()
        for nbr in (left, right):
            pl.semaphore_signal(barrier_sem, inc=1, device_id=(nbr,),
                                device_id_type=pl.DeviceIdType.MESH)
        pl.semaphore_wait(barrier_sem, 2)

        wbf_ref[...] = w_ref[...].astype(jnp.bfloat16)

        def partial(c, lo):
            xa = x_ref[pl.ds(c * m_per, m_per), :].astype(jnp.bfloat16)
            wslice = wbf_ref[:, :nh] if lo else wbf_ref[:, nh:]
            return jnp.dot(xa, wslice, preferred_element_type=jnp.float32)

        commR[0, :, :] = partial(left, True).astype(jnp.bfloat16)
        commL[0, :, :] = partial(right, False).astype(jnp.bfloat16)

        for s in range(N_DEV - 1):
            snd, rcv = s % 2, (s + 1) % 2
            if s >= 1:
                pl.semaphore_wait(credits.at[0], 1)
                pl.semaphore_wait(credits.at[1], 1)
            rdmaR = pltpu.make_async_remote_copy(
                src_ref=commR.at[snd], dst_ref=commR.at[rcv],
                send_sem=ssR.at[snd], recv_sem=rsR.at[rcv],
                device_id=(right,), device_id_type=pl.DeviceIdType.MESH,
            )
            rdmaL = pltpu.make_async_remote_copy(
                src_ref=commL.at[snd], dst_ref=commL.at[rcv],
                send_sem=ssL.at[snd], recv_sem=rsL.at[rcv],
                device_id=(left,), device_id_type=pl.DeviceIdType.MESH,
            )
            rdmaR.start()
            rdmaL.start()
            cR = lax.rem(my + 2 * N_DEV - 2 - s, N_DEV)
            cL = lax.rem(my + 2 + s, N_DEV)
            pbR[...] = partial(cR, True)
            pbL[...] = partial(cL, False)
            rdmaR.wait()
            rdmaL.wait()
            if s < N_DEV - 2:
                pl.semaphore_signal(credits.at[0], inc=1, device_id=(left,),
                                    device_id_type=pl.DeviceIdType.MESH)
                pl.semaphore_signal(credits.at[1], inc=1, device_id=(right,),
                                    device_id_type=pl.DeviceIdType.MESH)
                commR[rcv, :, :] = (commR[rcv, :, :] + pbR[...]).astype(jnp.bfloat16)
                commL[rcv, :, :] = (commL[rcv, :, :] + pbL[...]).astype(jnp.bfloat16)
            else:
                scale = sx_ref[0] * sw_ref[0]
                out_ref[:, :nh] = jnp.maximum((commR[rcv, :, :] + pbR[...]) * scale, 0.0)
                out_ref[:, nh:] = jnp.maximum((commL[rcv, :, :] + pbL[...]) * scale, 0.0)

    return pl.pallas_call(
        body,
        out_shape=jax.ShapeDtypeStruct((m_per, n), jnp.float32),
        in_specs=[
            pl.BlockSpec(memory_space=pltpu.VMEM),
            pl.BlockSpec(memory_space=pltpu.VMEM),
            pl.BlockSpec(memory_space=pltpu.SMEM),
            pl.BlockSpec(memory_space=pltpu.SMEM),
        ],
        out_specs=pl.BlockSpec(memory_space=pltpu.VMEM),
        scratch_shapes=[
            pltpu.VMEM((k_per, n), jnp.bfloat16),
            pltpu.VMEM((2, m_per, nh), jnp.bfloat16),
            pltpu.VMEM((2, m_per, nh), jnp.bfloat16),
            pltpu.VMEM((m_per, nh), jnp.float32),
            pltpu.VMEM((m_per, nh), jnp.float32),
            pltpu.SemaphoreType.DMA((2,)),
            pltpu.SemaphoreType.DMA((2,)),
            pltpu.SemaphoreType.DMA((2,)),
            pltpu.SemaphoreType.DMA((2,)),
            pltpu.SemaphoreType.REGULAR((2,)),
        ],
        compiler_params=pltpu.CompilerParams(collective_id=0),
    )(x, w_mat, scale_x, scale_w)


# device time: 361199 ns/iter; 3.9860x vs baseline; 1.1859x over previous
import jax
import jax.numpy as jnp
from jax import lax
from jax.experimental import pallas as pl
from jax.experimental.pallas import tpu as pltpu

N_DEV = 16
N_LANE = 4


def kernel(x, w_mat, scale_x, scale_w):
    k_full, k_per = x.shape
    _, n = w_mat.shape
    m_per = k_full // N_DEV
    nq = n // N_LANE
    lane_dir = (+1, +1, -1, -1)

    def body(x_ref, w_ref, sx_ref, sw_ref, out_ref, wbf_ref,
             c0, c1, c2, c3, p0, p1, p2, p3,
             ss0, rs0, ss1, rs1, ss2, rs2, ss3, rs3, credits):
        comm = (c0, c1, c2, c3)
        pb = (p0, p1, p2, p3)
        ss = (ss0, ss1, ss2, ss3)
        rs = (rs0, rs1, rs2, rs3)

        my = lax.axis_index("i")
        left = lax.rem(my + N_DEV - 1, N_DEV)
        right = lax.rem(my + 1, N_DEV)

        barrier_sem = pltpu.get_barrier_semaphore()
        for nbr in (left, right):
            pl.semaphore_signal(barrier_sem, inc=1, device_id=(nbr,),
                                device_id_type=pl.DeviceIdType.MESH)
        pl.semaphore_wait(barrier_sem, 2)

        wbf_ref[...] = w_ref[...].astype(jnp.bfloat16)

        def partial(c, ri):
            xa = x_ref[pl.ds(c * m_per, m_per), :].astype(jnp.bfloat16)
            return jnp.dot(xa, wbf_ref[:, ri * nq:(ri + 1) * nq],
                           preferred_element_type=jnp.float32)

        def arrive_chunk(ri, t):
            if lane_dir[ri] > 0:
                return lax.rem(my + 2 * N_DEV - 2 - t, N_DEV)
            return lax.rem(my + 2 + t, N_DEV)

        def make_rdma(ri, t):
            snd, rcv = t % 2, (t + 1) % 2
            dst = right if lane_dir[ri] > 0 else left
            return pltpu.make_async_remote_copy(
                src_ref=comm[ri].at[snd], dst_ref=comm[ri].at[rcv],
                send_sem=ss[ri].at[snd], recv_sem=rs[ri].at[rcv],
                device_id=(dst,), device_id_type=pl.DeviceIdType.MESH,
            )

        for ri in range(N_LANE):
            seed = left if lane_dir[ri] > 0 else right
            comm[ri][0, :, :] = partial(seed, ri).astype(jnp.bfloat16)
        for ri in range(N_LANE):
            make_rdma(ri, 0).start()

        scale = sx_ref[0] * sw_ref[0]

        for t in range(N_DEV - 1):
            rcv = (t + 1) % 2
            for ri in range(N_LANE):
                pb[ri][...] = partial(arrive_chunk(ri, t), ri)
            for ri in range(N_LANE):
                upstream = left if lane_dir[ri] > 0 else right
                rdma = make_rdma(ri, t)
                rdma.wait()
                if t < N_DEV - 2:
                    comm[ri][rcv, :, :] = (
                        comm[ri][rcv, :, :] + pb[ri][...]
                    ).astype(jnp.bfloat16)
                    pl.semaphore_signal(credits.at[ri], inc=1,
                                        device_id=(upstream,),
                                        device_id_type=pl.DeviceIdType.MESH)
                    pl.semaphore_wait(credits.at[ri], 1)
                    make_rdma(ri, t + 1).start()
                else:
                    out_ref[:, ri * nq:(ri + 1) * nq] = jnp.maximum(
                        (comm[ri][rcv, :, :] + pb[ri][...]) * scale, 0.0)

    return pl.pallas_call(
        body,
        out_shape=jax.ShapeDtypeStruct((m_per, n), jnp.float32),
        in_specs=[
            pl.BlockSpec(memory_space=pltpu.VMEM),
            pl.BlockSpec(memory_space=pltpu.VMEM),
            pl.BlockSpec(memory_space=pltpu.SMEM),
            pl.BlockSpec(memory_space=pltpu.SMEM),
        ],
        out_specs=pl.BlockSpec(memory_space=pltpu.VMEM),
        scratch_shapes=[
            pltpu.VMEM((k_per, n), jnp.bfloat16),
            *[pltpu.VMEM((2, m_per, n // N_LANE), jnp.bfloat16)
              for _ in range(N_LANE)],
            *[pltpu.VMEM((m_per, n // N_LANE), jnp.float32)
              for _ in range(N_LANE)],
            *[pltpu.SemaphoreType.DMA((2,))
              for _ in range(2 * N_LANE)],
            pltpu.SemaphoreType.REGULAR((N_LANE,)),
        ],
        compiler_params=pltpu.CompilerParams(collective_id=0),
    )(x, w_mat, scale_x, scale_w)
